# baseline (device time: 28362 ns/iter reference)
import jax
import jax.numpy as jnp
from jax import lax
from jax.experimental import pallas as pl
from jax.experimental.pallas import tpu as pltpu

N_DEV = 8
B, SQ, SKV, HL, DH = 2, 128, 128, 4, 64
DM = 512
DB = HL * DH


def kernel(x, Wq, K_ext, V_ext, Wo):
    Kf = K_ext.reshape(B, SKV, HL * DH)
    Vf = V_ext.reshape(B, SKV, HL * DH)

    def body(x_ref, wq_hbm, k_ref, v_ref, wo_hbm, out_ref,
             wq_vmem, wo_vmem, send_ref, recv_ref,
             load_sems, send_sems, recv_sems):
        my = lax.axis_index("i")
        partners = [lax.bitwise_xor(my, c) for c in (1, 3, 4)]

        wq_dma = pltpu.make_async_copy(
            wq_hbm.at[:, pl.ds(my * DB, DB)], wq_vmem, load_sems.at[0])
        wo_dma = pltpu.make_async_copy(
            wo_hbm.at[pl.ds(my * DB, DB), :], wo_vmem, load_sems.at[1])
        wq_dma.start()
        wo_dma.start()

        barrier = pltpu.get_barrier_semaphore()
        for nbr in partners:
            pl.semaphore_signal(barrier, inc=1, device_id=(nbr,),
                                device_id_type=pl.DeviceIdType.MESH)
        pl.semaphore_wait(barrier, len(partners))

        xb = x_ref[...].reshape(B * SQ, DM).astype(jnp.bfloat16)
        wq_dma.wait()
        q = lax.dot(xb, wq_vmem[...].astype(jnp.bfloat16),
                    preferred_element_type=jnp.float32).astype(jnp.bfloat16)
        ctx_rows = []
        for b in range(B):
            kb = k_ref[b].astype(jnp.bfloat16)
            vb = v_ref[b].astype(jnp.bfloat16)
            head_cols = []
            for h in range(HL):
                qbh = q[b * SQ:(b + 1) * SQ, h * DH:(h + 1) * DH]
                kbh = kb[:, h * DH:(h + 1) * DH]
                vbh = vb[:, h * DH:(h + 1) * DH]
                s = lax.dot_general(
                    qbh, kbh, (((1,), (1,)), ((), ())),
                    preferred_element_type=jnp.float32) * 0.125
                s = s - jnp.max(s, axis=1, keepdims=True)
                w = jnp.exp(s)
                w = w / jnp.sum(w, axis=1, keepdims=True)
                ctx = lax.dot(w.astype(jnp.bfloat16), vbh,
                              preferred_element_type=jnp.float32)
                head_cols.append(ctx.astype(jnp.bfloat16))
            ctx_rows.append(jnp.concatenate(head_cols, axis=1))
        ctx_all = jnp.concatenate(ctx_rows, axis=0)
        wo_dma.wait()
        partial = lax.dot(ctx_all, wo_vmem[...].astype(jnp.bfloat16),
                          preferred_element_type=jnp.float32)

        acc = partial
        for s, partner in enumerate(partners):
            send_ref[s, :, :] = acc.astype(jnp.bfloat16)
            rdma = pltpu.make_async_remote_copy(
                src_ref=send_ref.at[s],
                dst_ref=recv_ref.at[s],
                send_sem=send_sems.at[s],
                recv_sem=recv_sems.at[s],
                device_id=(partner,),
                device_id_type=pl.DeviceIdType.MESH,
            )
            rdma.start()
            rdma.wait()
            acc = acc + recv_ref[s, :, :].astype(jnp.float32)
        out_ref[...] = acc.reshape(B, SQ, DM)

    return pl.pallas_call(
        body,
        out_shape=jax.ShapeDtypeStruct((B, SQ, DM), jnp.float32),
        in_specs=[
            pl.BlockSpec(memory_space=pltpu.VMEM),
            pl.BlockSpec(memory_space=pl.ANY),
            pl.BlockSpec(memory_space=pltpu.VMEM),
            pl.BlockSpec(memory_space=pltpu.VMEM),
            pl.BlockSpec(memory_space=pl.ANY),
        ],
        out_specs=pl.BlockSpec(memory_space=pltpu.VMEM),
        scratch_shapes=[
            pltpu.VMEM((DM, DB), jnp.float32),
            pltpu.VMEM((DB, DM), jnp.float32),
            pltpu.VMEM((3, B * SQ, DM), jnp.bfloat16),
            pltpu.VMEM((3, B * SQ, DM), jnp.bfloat16),
            pltpu.SemaphoreType.DMA((2,)),
            pltpu.SemaphoreType.DMA((3,)),
            pltpu.SemaphoreType.DMA((3,)),
        ],
        compiler_params=pltpu.CompilerParams(collective_id=0),
    )(x, Wq, Kf, Vf, Wo)


# device time: 25525 ns/iter; 1.1111x vs baseline; 1.1111x over previous
import jax
import jax.numpy as jnp
from jax import lax
from jax.experimental import pallas as pl
from jax.experimental.pallas import tpu as pltpu

N_DEV = 8
B, SQ, SKV, HL, DH = 2, 128, 128, 4, 64
DM = 512
DB = HL * DH


def kernel(x, Wq, K_ext, V_ext, Wo):
    idx = lax.axis_index("i")
    Wq_sl = lax.dynamic_slice_in_dim(Wq, idx * DB, DB, axis=1).astype(jnp.bfloat16)
    Wo_sl = lax.dynamic_slice_in_dim(Wo, idx * DB, DB, axis=0).astype(jnp.bfloat16)
    xb = x.reshape(B * SQ, DM).astype(jnp.bfloat16)
    Kb = K_ext.reshape(B, SKV, HL * DH).astype(jnp.bfloat16)
    Vb = V_ext.reshape(B, SKV, HL * DH).astype(jnp.bfloat16)

    def body(x_ref, wq_ref, k_ref, v_ref, wo_ref, out_ref,
             send_ref, recv_ref, send_sems, recv_sems):
        my = lax.axis_index("i")
        partners = [lax.bitwise_xor(my, c) for c in (1, 3, 4)]

        barrier = pltpu.get_barrier_semaphore()
        for nbr in partners:
            pl.semaphore_signal(barrier, inc=1, device_id=(nbr,),
                                device_id_type=pl.DeviceIdType.MESH)
        pl.semaphore_wait(barrier, len(partners))

        q = lax.dot(x_ref[...], wq_ref[...],
                    preferred_element_type=jnp.float32).astype(jnp.bfloat16)
        ctx_rows = []
        for b in range(B):
            kb = k_ref[b]
            vb = v_ref[b]
            head_cols = []
            for h in range(HL):
                qbh = q[b * SQ:(b + 1) * SQ, h * DH:(h + 1) * DH]
                kbh = kb[:, h * DH:(h + 1) * DH]
                vbh = vb[:, h * DH:(h + 1) * DH]
                s = lax.dot_general(
                    qbh, kbh, (((1,), (1,)), ((), ())),
                    preferred_element_type=jnp.float32) * 0.125
                s = s - jnp.max(s, axis=1, keepdims=True)
                w = jnp.exp(s)
                w = w / jnp.sum(w, axis=1, keepdims=True)
                ctx = lax.dot(w.astype(jnp.bfloat16), vbh,
                              preferred_element_type=jnp.float32)
                head_cols.append(ctx.astype(jnp.bfloat16))
            ctx_rows.append(jnp.concatenate(head_cols, axis=1))
        ctx_all = jnp.concatenate(ctx_rows, axis=0)
        partial = lax.dot(ctx_all, wo_ref[...],
                          preferred_element_type=jnp.float32)

        acc = partial
        for s, partner in enumerate(partners):
            send_ref[s, :, :] = acc.astype(jnp.bfloat16)
            rdma = pltpu.make_async_remote_copy(
                src_ref=send_ref.at[s],
                dst_ref=recv_ref.at[s],
                send_sem=send_sems.at[s],
                recv_sem=recv_sems.at[s],
                device_id=(partner,),
                device_id_type=pl.DeviceIdType.MESH,
            )
            rdma.start()
            rdma.wait()
            acc = acc + recv_ref[s, :, :].astype(jnp.float32)
        out_ref[...] = acc.reshape(B, SQ, DM)

    return pl.pallas_call(
        body,
        out_shape=jax.ShapeDtypeStruct((B, SQ, DM), jnp.float32),
        in_specs=[pl.BlockSpec(memory_space=pltpu.VMEM)] * 5,
        out_specs=pl.BlockSpec(memory_space=pltpu.VMEM),
        scratch_shapes=[
            pltpu.VMEM((3, B * SQ, DM), jnp.bfloat16),
            pltpu.VMEM((3, B * SQ, DM), jnp.bfloat16),
            pltpu.SemaphoreType.DMA((3,)),
            pltpu.SemaphoreType.DMA((3,)),
        ],
        compiler_params=pltpu.CompilerParams(collective_id=0),
    )(xb, Wq_sl, Kb, Vb, Wo_sl)


# device time: 21356 ns/iter; 1.3281x vs baseline; 1.1952x over previous
import jax
import jax.numpy as jnp
from jax import lax
from jax.experimental import pallas as pl
from jax.experimental.pallas import tpu as pltpu

N_DEV = 8
B, SQ, SKV, HL, DH = 2, 128, 128, 4, 64
DM = 512
DB = HL * DH

XOR_A = (1, 3, 4)
XOR_B = (3, 4, 1)


def kernel(x, Wq, K_ext, V_ext, Wo):
    idx = lax.axis_index("i")
    Wq_sl = lax.dynamic_slice_in_dim(Wq, idx * DB, DB, axis=1).astype(jnp.bfloat16)
    Wo_sl = lax.dynamic_slice_in_dim(Wo, idx * DB, DB, axis=0).astype(jnp.bfloat16)
    Kf = K_ext.reshape(B, SKV, HL * DH)
    Vf = V_ext.reshape(B, SKV, HL * DH)

    def body(x_ref, wq_ref, k_ref, v_ref, wo_ref, out_ref,
             send_ref, recv_ref, send_sems, recv_sems):
        my = lax.axis_index("i")
        partners_a = [lax.bitwise_xor(my, c) for c in XOR_A]
        partners_b = [lax.bitwise_xor(my, c) for c in XOR_B]

        barrier = pltpu.get_barrier_semaphore()
        for nbr in partners_a:
            pl.semaphore_signal(barrier, inc=1, device_id=(nbr,),
                                device_id_type=pl.DeviceIdType.MESH)
        pl.semaphore_wait(barrier, len(partners_a))

        accs = []
        for b in range(B):
            xb = x_ref[b].astype(jnp.bfloat16)
            kb = k_ref[b].astype(jnp.bfloat16)
            vb = v_ref[b].astype(jnp.bfloat16)
            qb = lax.dot(xb, wq_ref[...],
                         preferred_element_type=jnp.float32).astype(jnp.bfloat16)
            head_cols = []
            for h in range(HL):
                qbh = qb[:, h * DH:(h + 1) * DH]
                kbh = kb[:, h * DH:(h + 1) * DH]
                vbh = vb[:, h * DH:(h + 1) * DH]
                s = lax.dot_general(
                    qbh, kbh, (((1,), (1,)), ((), ())),
                    preferred_element_type=jnp.float32) * 0.125
                s = s - jnp.max(s, axis=1, keepdims=True)
                w = jnp.exp(s)
                w = w / jnp.sum(w, axis=1, keepdims=True)
                ctx = lax.dot(w.astype(jnp.bfloat16), vbh,
                              preferred_element_type=jnp.float32)
                head_cols.append(ctx.astype(jnp.bfloat16))
            ctx_b = jnp.concatenate(head_cols, axis=1)
            accs.append(lax.dot(ctx_b, wo_ref[...],
                                preferred_element_type=jnp.float32))

        acc_a, acc_b = accs
        for s in range(3):
            send_ref[s, 0:SQ, :] = acc_a.astype(jnp.bfloat16)
            send_ref[s, SQ:2 * SQ, :] = acc_b.astype(jnp.bfloat16)
            rdma_a = pltpu.make_async_remote_copy(
                src_ref=send_ref.at[s, pl.ds(0, SQ)],
                dst_ref=recv_ref.at[s, pl.ds(0, SQ)],
                send_sem=send_sems.at[s, 0],
                recv_sem=recv_sems.at[s, 0],
                device_id=(partners_a[s],),
                device_id_type=pl.DeviceIdType.MESH,
            )
            rdma_b = pltpu.make_async_remote_copy(
                src_ref=send_ref.at[s, pl.ds(SQ, SQ)],
                dst_ref=recv_ref.at[s, pl.ds(SQ, SQ)],
                send_sem=send_sems.at[s, 1],
                recv_sem=recv_sems.at[s, 1],
                device_id=(partners_b[s],),
                device_id_type=pl.DeviceIdType.MESH,
            )
            rdma_a.start()
            rdma_b.start()
            rdma_a.wait()
            acc_a = acc_a + recv_ref[s, 0:SQ, :].astype(jnp.float32)
            rdma_b.wait()
            acc_b = acc_b + recv_ref[s, SQ:2 * SQ, :].astype(jnp.float32)
        out_ref[0] = acc_a
        out_ref[1] = acc_b

    return pl.pallas_call(
        body,
        out_shape=jax.ShapeDtypeStruct((B, SQ, DM), jnp.float32),
        in_specs=[pl.BlockSpec(memory_space=pltpu.VMEM)] * 5,
        out_specs=pl.BlockSpec(memory_space=pltpu.VMEM),
        scratch_shapes=[
            pltpu.VMEM((3, B * SQ, DM), jnp.bfloat16),
            pltpu.VMEM((3, B * SQ, DM), jnp.bfloat16),
            pltpu.SemaphoreType.DMA((3, 2)),
            pltpu.SemaphoreType.DMA((3, 2)),
        ],
        compiler_params=pltpu.CompilerParams(collective_id=0),
    )(x, Wq_sl, Kf, Vf, Wo_sl)


# device time: 21347 ns/iter; 1.3286x vs baseline; 1.0004x over previous
import jax
import jax.numpy as jnp
from jax import lax
from jax.experimental import pallas as pl
from jax.experimental.pallas import tpu as pltpu

N_DEV = 8
B, SQ, SKV, HL, DH = 2, 128, 128, 4, 64
DM = 512
DB = HL * DH

XOR_A = (1, 3, 4)
XOR_B = (3, 4, 1)


def kernel(x, Wq, K_ext, V_ext, Wo):
    idx = lax.axis_index("i")
    Wq_sl = lax.dynamic_slice_in_dim(Wq, idx * DB, DB, axis=1).astype(jnp.bfloat16)
    Wo_sl = lax.dynamic_slice_in_dim(Wo, idx * DB, DB, axis=0).astype(jnp.bfloat16)
    Kf = K_ext.reshape(B, SKV, HL * DH)
    Vf = V_ext.reshape(B, SKV, HL * DH)

    def body(x_ref, wq_ref, k_ref, v_ref, wo_ref, out_ref,
             send_ref, recv_ref, send_sems, recv_sems):
        my = lax.axis_index("i")
        partners_a = [lax.bitwise_xor(my, c) for c in XOR_A]
        partners_b = [lax.bitwise_xor(my, c) for c in XOR_B]

        barrier = pltpu.get_barrier_semaphore()
        for nbr in partners_a:
            pl.semaphore_signal(barrier, inc=1, device_id=(nbr,),
                                device_id_type=pl.DeviceIdType.MESH)

        def attention(b):
            xb = x_ref[b].astype(jnp.bfloat16)
            kb = k_ref[b].astype(jnp.bfloat16)
            vb = v_ref[b].astype(jnp.bfloat16)
            qb = lax.dot(xb, wq_ref[...],
                         preferred_element_type=jnp.float32).astype(jnp.bfloat16)
            head_cols = []
            for h in range(HL):
                qbh = qb[:, h * DH:(h + 1) * DH]
                kbh = kb[:, h * DH:(h + 1) * DH]
                vbh = vb[:, h * DH:(h + 1) * DH]
                s = lax.dot_general(
                    qbh, kbh, (((1,), (1,)), ((), ())),
                    preferred_element_type=jnp.float32) * 0.125
                s = s - jnp.max(s, axis=1, keepdims=True)
                w = jnp.exp(s)
                w = w / jnp.sum(w, axis=1, keepdims=True)
                ctx = lax.dot(w.astype(jnp.bfloat16), vbh,
                              preferred_element_type=jnp.float32)
                head_cols.append(ctx.astype(jnp.bfloat16))
            ctx_b = jnp.concatenate(head_cols, axis=1)
            return lax.dot(ctx_b, wo_ref[...],
                           preferred_element_type=jnp.float32)

        def mk(s, half, partner):
            return pltpu.make_async_remote_copy(
                src_ref=send_ref.at[s, pl.ds(half * SQ, SQ)],
                dst_ref=recv_ref.at[s, pl.ds(half * SQ, SQ)],
                send_sem=send_sems.at[s, half],
                recv_sem=recv_sems.at[s, half],
                device_id=(partner,),
                device_id_type=pl.DeviceIdType.MESH,
            )

        acc_a = attention(0)
        send_ref[0, 0:SQ, :] = acc_a.astype(jnp.bfloat16)
        pl.semaphore_wait(barrier, len(partners_a))
        rdma = {(0, 0): mk(0, 0, partners_a[0])}
        rdma[(0, 0)].start()

        acc_b = attention(1)
        send_ref[0, SQ:2 * SQ, :] = acc_b.astype(jnp.bfloat16)
        rdma[(0, 1)] = mk(0, 1, partners_b[0])
        rdma[(0, 1)].start()

        for s in range(3):
            rdma[(s, 0)].wait()
            acc_a = acc_a + recv_ref[s, 0:SQ, :].astype(jnp.float32)
            if s < 2:
                send_ref[s + 1, 0:SQ, :] = acc_a.astype(jnp.bfloat16)
                rdma[(s + 1, 0)] = mk(s + 1, 0, partners_a[s + 1])
                rdma[(s + 1, 0)].start()
            rdma[(s, 1)].wait()
            acc_b = acc_b + recv_ref[s, SQ:2 * SQ, :].astype(jnp.float32)
            if s < 2:
                send_ref[s + 1, SQ:2 * SQ, :] = acc_b.astype(jnp.bfloat16)
                rdma[(s + 1, 1)] = mk(s + 1, 1, partners_b[s + 1])
                rdma[(s + 1, 1)].start()
        out_ref[0] = acc_a
        out_ref[1] = acc_b

    return pl.pallas_call(
        body,
        out_shape=jax.ShapeDtypeStruct((B, SQ, DM), jnp.float32),
        in_specs=[pl.BlockSpec(memory_space=pltpu.VMEM)] * 5,
        out_specs=pl.BlockSpec(memory_space=pltpu.VMEM),
        scratch_shapes=[
            pltpu.VMEM((3, B * SQ, DM), jnp.bfloat16),
            pltpu.VMEM((3, B * SQ, DM), jnp.bfloat16),
            pltpu.SemaphoreType.DMA((3, 2)),
            pltpu.SemaphoreType.DMA((3, 2)),
        ],
        compiler_params=pltpu.CompilerParams(collective_id=0),
    )(x, Wq_sl, Kf, Vf, Wo_sl)


# device time: 18681 ns/iter; 1.5182x vs baseline; 1.1427x over previous
import jax
import jax.numpy as jnp
from jax import lax
from jax.experimental import pallas as pl
from jax.experimental.pallas import tpu as pltpu

N_DEV = 8
B, SQ, SKV, HL, DH = 2, 128, 128, 4, 64
DM = 512
DB = HL * DH

XOR_AXES = (1, 3, 4)
ROWS = B * SQ
CHUNKS = ((0, 88), (88, 88), (176, 80))


def kernel(x, Wq, K_ext, V_ext, Wo):
    idx = lax.axis_index("i")
    Wq_sl = lax.dynamic_slice_in_dim(Wq, idx * DB, DB, axis=1).astype(jnp.bfloat16)
    Wo_sl = lax.dynamic_slice_in_dim(Wo, idx * DB, DB, axis=0).astype(jnp.bfloat16)
    Kf = K_ext.reshape(B, SKV, HL * DH)
    Vf = V_ext.reshape(B, SKV, HL * DH)

    def body(x_ref, wq_ref, k_ref, v_ref, wo_ref, out_ref,
             send_ref, recv_ref, send_sems, recv_sems):
        my = lax.axis_index("i")
        nbrs = [lax.bitwise_xor(my, c) for c in XOR_AXES]

        barrier = pltpu.get_barrier_semaphore()
        for nbr in nbrs:
            pl.semaphore_signal(barrier, inc=1, device_id=(nbr,),
                                device_id_type=pl.DeviceIdType.MESH)

        xb = x_ref[...].reshape(ROWS, DM).astype(jnp.bfloat16)
        q = lax.dot(xb, wq_ref[...],
                    preferred_element_type=jnp.float32).astype(jnp.bfloat16)
        ctx_rows = []
        for b in range(B):
            kb = k_ref[b].astype(jnp.bfloat16)
            vb = v_ref[b].astype(jnp.bfloat16)
            head_cols = []
            for h in range(HL):
                qbh = q[b * SQ:(b + 1) * SQ, h * DH:(h + 1) * DH]
                kbh = kb[:, h * DH:(h + 1) * DH]
                vbh = vb[:, h * DH:(h + 1) * DH]
                s = lax.dot_general(
                    qbh, kbh, (((1,), (1,)), ((), ())),
                    preferred_element_type=jnp.float32) * 0.125
                w = jnp.exp(s)
                recip = 1.0 / jnp.sum(w, axis=1, keepdims=True)
                ctx = lax.dot(w.astype(jnp.bfloat16), vbh,
                              preferred_element_type=jnp.float32)
                head_cols.append((ctx * recip).astype(jnp.bfloat16))
            ctx_rows.append(jnp.concatenate(head_cols, axis=1))
        ctx_all = jnp.concatenate(ctx_rows, axis=0)
        acc = lax.dot(ctx_all, wo_ref[...],
                      preferred_element_type=jnp.float32)

        pl.semaphore_wait(barrier, len(nbrs))
        for s in range(3):
            rdmas = []
            for p, (r0, rn) in enumerate(CHUNKS):
                send_ref[s, pl.ds(r0, rn), :] = acc[r0:r0 + rn, :].astype(jnp.bfloat16)
                rdma = pltpu.make_async_remote_copy(
                    src_ref=send_ref.at[s, pl.ds(r0, rn)],
                    dst_ref=recv_ref.at[s, pl.ds(r0, rn)],
                    send_sem=send_sems.at[s, p],
                    recv_sem=recv_sems.at[s, p],
                    device_id=(nbrs[(s + p) % 3],),
                    device_id_type=pl.DeviceIdType.MESH,
                )
                rdma.start()
                rdmas.append(rdma)
            parts = []
            for p, (r0, rn) in enumerate(CHUNKS):
                rdmas[p].wait()
                parts.append(acc[r0:r0 + rn, :]
                             + recv_ref[s, pl.ds(r0, rn), :].astype(jnp.float32))
            acc = jnp.concatenate(parts, axis=0)
        out_ref[...] = acc.reshape(B, SQ, DM)

    return pl.pallas_call(
        body,
        out_shape=jax.ShapeDtypeStruct((B, SQ, DM), jnp.float32),
        in_specs=[pl.BlockSpec(memory_space=pltpu.VMEM)] * 5,
        out_specs=pl.BlockSpec(memory_space=pltpu.VMEM),
        scratch_shapes=[
            pltpu.VMEM((3, ROWS, DM), jnp.bfloat16),
            pltpu.VMEM((3, ROWS, DM), jnp.bfloat16),
            pltpu.SemaphoreType.DMA((3, 3)),
            pltpu.SemaphoreType.DMA((3, 3)),
        ],
        compiler_params=pltpu.CompilerParams(collective_id=0),
    )(x, Wq_sl, Kf, Vf, Wo_sl)
